# baseline (speedup 1.0000x reference)
import functools

import jax
import jax.numpy as jnp
from jax import lax
from jax.experimental import pallas as pl
from jax.experimental.pallas import tpu as pltpu
from jax.experimental.pallas import tpu_sc as plsc

N = 10000
D = 128
G = 64
T = 10
L = 3
RS = 1.0 / (1.0 + 1e-5) ** 0.5

NC = 2
NS = 16
NW = NC * NS
CH = 128
C = 80
C0 = 80
C1 = 80
HC = 16
TOTC = NS * (C0 + C1)
EPAD = NW * C * CH
NPAD = 10112
RPS = NPAD // NS

R = 1000
GRID = N // R

_mesh = plsc.VectorSubcoreMesh(core_axis_name="c", subcore_axis_name="s")


@functools.partial(
    pl.kernel,
    out_type=jax.ShapeDtypeStruct((2 * NPAD, D), jnp.float32),
    mesh=_mesh,
    scratch_types=[
        pltpu.VMEM((C, CH), jnp.int32),
        pltpu.VMEM((CH, D), jnp.float32),
        pltpu.VMEM_SHARED((NPAD, D), jnp.float32),
    ],
)
def _sc_degree(dst_hbm, ones_hbm, zdeg_hbm, out_hbm, dst_v, ones_v, deg_sh):
    c = lax.axis_index("c")
    s = lax.axis_index("s")
    w = c * NS + s
    pltpu.sync_copy(zdeg_hbm, deg_sh.at[pl.ds(s * RPS, RPS)])
    pltpu.sync_copy(dst_hbm.at[w], dst_v)
    pltpu.sync_copy(ones_hbm, ones_v)
    plsc.subcore_barrier()

    def body(j, carry):
        pltpu.sync_copy(ones_v, deg_sh.at[dst_v.at[j]], add=True)
        return carry

    lax.fori_loop(0, C, body, 0)
    plsc.subcore_barrier()
    pltpu.sync_copy(deg_sh.at[pl.ds(s * RPS, RPS)],
                    out_hbm.at[pl.ds(c * NPAD + s * RPS, RPS)])


@functools.partial(
    pl.kernel,
    out_type=jax.ShapeDtypeStruct((2 * NPAD, D), jnp.float32),
    mesh=_mesh,
    scratch_types=[
        pltpu.VMEM((HC, CH), jnp.int32),
        pltpu.VMEM((HC, CH), jnp.int32),
        pltpu.VMEM((CH, D), jnp.float32),
        pltpu.VMEM((CH, D), jnp.float32),
        pltpu.VMEM_SHARED((NPAD, D), jnp.float32),
        pltpu.SemaphoreType.DMA,
        pltpu.SemaphoreType.DMA,
    ],
)
def _sc_scatter(hws_hbm, src_hbm, dst_hbm, zacc_hbm, out_hbm,
                src_v, dst_v, buf0, buf1, acc_sh, sem0, sem1):
    c = lax.axis_index("c")
    s = lax.axis_index("s")
    pltpu.sync_copy(zacc_hbm, acc_sh.at[pl.ds(s * RPS, RPS)])
    plsc.subcore_barrier()

    cc = jnp.where(c == 0, C0, C1)
    base = c * (NS * C0) + s * cc
    nph = jnp.where(c == 0, C0 // HC, C1 // HC)
    bufs = (buf0, buf1)
    sems = (sem0, sem1)

    def phase(p, carry):
        row0 = pl.multiple_of(base + p * HC, 8)
        pltpu.sync_copy(src_hbm.at[pl.ds(row0, HC)], src_v)
        pltpu.sync_copy(dst_hbm.at[pl.ds(row0, HC)], dst_v)
        pltpu.async_copy(hws_hbm.at[src_v.at[0]], buf0, sem0)
        pltpu.async_copy(hws_hbm.at[src_v.at[1]], buf1, sem1)

        def body(i, carry2):
            for b in range(2):
                j = 2 * i + b
                buf, sem = bufs[b], sems[b]
                pltpu.make_async_copy(hws_hbm.at[pl.ds(0, CH)], buf,
                                      sem).wait()
                pltpu.sync_copy(buf, acc_sh.at[dst_v.at[j]], add=True)

                @pl.when(j + 2 < HC)
                def _():
                    pltpu.async_copy(hws_hbm.at[src_v.at[j + 2]], buf, sem)
            return carry2

        lax.fori_loop(0, HC // 2, body, 0)
        return carry

    lax.fori_loop(0, nph, phase, 0)
    plsc.subcore_barrier()
    pltpu.sync_copy(acc_sh.at[pl.ds(s * RPS, RPS)],
                    out_hbm.at[pl.ds(c * NPAD + s * RPS, RPS)])


def _tca_body(degp_ref, x_ref, embW_ref, embb_ref, W0_ref, dinv_ref, hws_ref):
    degp = degp_ref[...]
    deg = degp[0][:, :1] + degp[1][:, :1] + 1.0
    dinv = lax.rsqrt(deg)
    h0 = x_ref[...] * embW_ref[...] + embb_ref[...]
    hw0 = jnp.dot(h0, W0_ref[...], preferred_element_type=jnp.float32)
    dinv_ref[...] = dinv
    hws_ref[...] = hw0 * dinv


def _tcb_body(aggp_ref, hwsp_ref, dinv_ref, b_ref, g_ref, bt_ref, Wn_ref,
              out_ref):
    aggp = aggp_ref[...]
    dinv = dinv_ref[...]
    agg = (aggp[0] + aggp[1] + hwsp_ref[...]) * dinv
    hh = (agg + b_ref[...]) * (g_ref[...] * RS) + bt_ref[...]
    h = jnp.maximum(hh, 0.0)
    out_ref[...] = jnp.dot(h, Wn_ref[...],
                           preferred_element_type=jnp.float32) * dinv


def _tcc_body(aggp_ref, hwsp_ref, dinv_ref, b_ref, g_ref, bt_ref, batch_ref,
              pW_ref, pb_ref, out_ref, s_sc, c_sc):
    i = pl.program_id(0)
    aggp = aggp_ref[...]
    agg = (aggp[0] + aggp[1] + hwsp_ref[...]) * dinv_ref[...]
    hh = (agg + b_ref[...]) * (g_ref[...] * RS) + bt_ref[...]
    br = batch_ref[0]
    gcol = lax.broadcasted_iota(jnp.int32, (G, 1), 0)
    m = (br == gcol).astype(jnp.float32)

    @pl.when(i == 0)
    def _():
        s_sc[...] = jnp.zeros((G, D), jnp.float32)
        c_sc[...] = jnp.zeros((G, 1), jnp.float32)

    s_sc[...] += jnp.dot(m, hh, preferred_element_type=jnp.float32)
    c_sc[...] += jnp.sum(m, axis=1, keepdims=True)

    @pl.when(i == GRID - 1)
    def _():
        hg = s_sc[...] / jnp.maximum(c_sc[...], 1.0)
        out_ref[...] = (jnp.dot(hg, pW_ref[...],
                                preferred_element_type=jnp.float32)
                        + pb_ref[...])


def _row_spec(w):
    return pl.BlockSpec((R, w), lambda i: (i, 0))


def _const_spec(shape):
    ndim = len(shape)
    return pl.BlockSpec(shape, lambda i: (0,) * ndim)


_AGGP_SPEC = pl.BlockSpec((2, R, D), lambda i: (0, i, 0))
_F32 = jnp.float32


def _tc_a(degp3, x, embW, embb2, W0):
    return pl.pallas_call(
        _tca_body,
        grid=(GRID,),
        in_specs=[_AGGP_SPEC,
                  _row_spec(1), _const_spec((1, D)), _const_spec((1, D)),
                  _const_spec((D, D))],
        out_specs=[_row_spec(1), _row_spec(D)],
        out_shape=[jax.ShapeDtypeStruct((N, 1), _F32),
                   jax.ShapeDtypeStruct((N, D), _F32)],
    )(degp3, x, embW, embb2, W0)


def _tc_b(aggp3, hwsp, dinv, b2, g2, bt2, Wn):
    return pl.pallas_call(
        _tcb_body,
        grid=(GRID,),
        in_specs=[_AGGP_SPEC, _row_spec(D), _row_spec(1),
                  _const_spec((1, D)), _const_spec((1, D)),
                  _const_spec((1, D)), _const_spec((D, D))],
        out_specs=_row_spec(D),
        out_shape=jax.ShapeDtypeStruct((N, D), _F32),
    )(aggp3, hwsp, dinv, b2, g2, bt2, Wn)


def _tc_c(aggp3, hwsp, dinv, b2, g2, bt2, batch3, pW, pb2):
    return pl.pallas_call(
        _tcc_body,
        grid=(GRID,),
        in_specs=[_AGGP_SPEC, _row_spec(D), _row_spec(1),
                  _const_spec((1, D)), _const_spec((1, D)),
                  _const_spec((1, D)),
                  pl.BlockSpec((1, 1, R), lambda i: (i, 0, 0)),
                  _const_spec((D, T)), _const_spec((1, T))],
        out_specs=pl.BlockSpec((G, T), lambda i: (0, 0)),
        out_shape=jax.ShapeDtypeStruct((G, T), _F32),
        scratch_shapes=[pltpu.VMEM((G, D), _F32), pltpu.VMEM((G, 1), _F32)],
    )(aggp3, hwsp, dinv, b2, g2, bt2, batch3, pW, pb2)


def kernel(x, edge_index, edge_attr, batch, emb_W, emb_b, conv_W, conv_b,
           bn_gamma, bn_beta, pred_W, pred_b):
    E = edge_index.shape[1]
    pad = EPAD - E
    pad_src = jnp.arange(pad, dtype=jnp.int32) % N
    src_f = jnp.concatenate([edge_index[0], pad_src]).reshape(TOTC, CH)
    pad_dst = N + jnp.arange(pad, dtype=jnp.int32) % (NPAD - N)
    dst_f = jnp.concatenate([edge_index[1], pad_dst]).reshape(TOTC, CH)
    dst_p = dst_f.reshape(NW, C, CH)
    ones128 = jnp.ones((CH, D), _F32)
    zacc = jnp.zeros((RPS, D), _F32)

    degp = _sc_degree(dst_p, ones128, zacc).reshape(2, NPAD, D)
    dinv, hws = _tc_a(degp, x, emb_W, emb_b.reshape(1, D), conv_W[0])

    for l in range(L - 1):
        aggp = _sc_scatter(hws, src_f, dst_f, zacc).reshape(2, NPAD, D)
        hws = _tc_b(aggp, hws, dinv, conv_b[l].reshape(1, D),
                    bn_gamma[l].reshape(1, D), bn_beta[l].reshape(1, D),
                    conv_W[l + 1])

    aggp = _sc_scatter(hws, src_f, dst_f, zacc).reshape(2, NPAD, D)
    return _tc_c(aggp, hws, dinv, conv_b[L - 1].reshape(1, D),
                 bn_gamma[L - 1].reshape(1, D), bn_beta[L - 1].reshape(1, D),
                 batch.reshape(GRID, 1, R), pred_W, pred_b.reshape(1, T))


# speedup vs baseline: 1.0520x; 1.0520x over previous
import functools

import jax
import jax.numpy as jnp
from jax import lax
from jax.experimental import pallas as pl
from jax.experimental.pallas import tpu as pltpu
from jax.experimental.pallas import tpu_sc as plsc

N = 10000
D = 128
G = 64
T = 10
L = 3
RS = 1.0 / (1.0 + 1e-5) ** 0.5

NC = 2
NS = 16
NW = NC * NS
CH = 128
C = 80
C0 = 80
C1 = 80
HC = 40
TOTC = NS * (C0 + C1)
EPAD = NW * C * CH
NPAD = 10112
RPS = NPAD // NS

R = 1000
GRID = N // R

_mesh = plsc.VectorSubcoreMesh(core_axis_name="c", subcore_axis_name="s")


@functools.partial(
    pl.kernel,
    out_type=jax.ShapeDtypeStruct((2 * NPAD, D), jnp.float32),
    mesh=_mesh,
    scratch_types=[
        pltpu.VMEM((C, CH), jnp.int32),
        pltpu.VMEM((CH, D), jnp.float32),
        pltpu.VMEM_SHARED((NPAD, D), jnp.float32),
    ],
)
def _sc_degree(dst_hbm, ones_hbm, zdeg_hbm, out_hbm, dst_v, ones_v, deg_sh):
    c = lax.axis_index("c")
    s = lax.axis_index("s")
    w = c * NS + s
    pltpu.sync_copy(zdeg_hbm, deg_sh.at[pl.ds(s * RPS, RPS)])
    pltpu.sync_copy(dst_hbm.at[w], dst_v)
    pltpu.sync_copy(ones_hbm, ones_v)
    plsc.subcore_barrier()

    def body(j, carry):
        pltpu.sync_copy(ones_v, deg_sh.at[dst_v.at[j]], add=True)
        return carry

    lax.fori_loop(0, C, body, 0)
    plsc.subcore_barrier()
    pltpu.sync_copy(deg_sh.at[pl.ds(s * RPS, RPS)],
                    out_hbm.at[pl.ds(c * NPAD + s * RPS, RPS)])


@functools.partial(
    pl.kernel,
    out_type=jax.ShapeDtypeStruct((2 * NPAD, D), jnp.float32),
    mesh=_mesh,
    scratch_types=[
        pltpu.VMEM((HC, CH), jnp.int32),
        pltpu.VMEM((HC, CH), jnp.int32),
        pltpu.VMEM((CH, D), jnp.float32),
        pltpu.VMEM((CH, D), jnp.float32),
        pltpu.VMEM_SHARED((NPAD, D), jnp.float32),
        pltpu.SemaphoreType.DMA,
        pltpu.SemaphoreType.DMA,
    ],
)
def _sc_scatter(hws_hbm, src_hbm, dst_hbm, zacc_hbm, out_hbm,
                src_v, dst_v, buf0, buf1, acc_sh, sem0, sem1):
    c = lax.axis_index("c")
    s = lax.axis_index("s")
    pltpu.sync_copy(zacc_hbm, acc_sh.at[pl.ds(s * RPS, RPS)])
    plsc.subcore_barrier()

    cc = jnp.where(c == 0, C0, C1)
    base = c * (NS * C0) + s * cc
    nph = jnp.where(c == 0, C0 // HC, C1 // HC)
    bufs = (buf0, buf1)
    sems = (sem0, sem1)

    def phase(p, carry):
        row0 = pl.multiple_of(base + p * HC, 8)
        pltpu.sync_copy(src_hbm.at[pl.ds(row0, HC)], src_v)
        pltpu.sync_copy(dst_hbm.at[pl.ds(row0, HC)], dst_v)
        pltpu.async_copy(hws_hbm.at[src_v.at[0]], buf0, sem0)
        pltpu.async_copy(hws_hbm.at[src_v.at[1]], buf1, sem1)

        def body(i, carry2):
            for b in range(2):
                j = 2 * i + b
                buf, sem = bufs[b], sems[b]
                pltpu.make_async_copy(hws_hbm.at[pl.ds(0, CH)], buf,
                                      sem).wait()
                pltpu.sync_copy(buf, acc_sh.at[dst_v.at[j]], add=True)

                @pl.when(j + 2 < HC)
                def _():
                    pltpu.async_copy(hws_hbm.at[src_v.at[j + 2]], buf, sem)
            return carry2

        lax.fori_loop(0, HC // 2, body, 0)
        return carry

    lax.fori_loop(0, nph, phase, 0)
    plsc.subcore_barrier()
    pltpu.sync_copy(acc_sh.at[pl.ds(s * RPS, RPS)],
                    out_hbm.at[pl.ds(c * NPAD + s * RPS, RPS)])


def _tca_body(degp_ref, x_ref, embW_ref, embb_ref, W0_ref, dinv_ref, hws_ref):
    degp = degp_ref[...]
    deg = degp[0][:, :1] + degp[1][:, :1] + 1.0
    dinv = lax.rsqrt(deg)
    h0 = x_ref[...] * embW_ref[...] + embb_ref[...]
    hw0 = jnp.dot(h0, W0_ref[...], preferred_element_type=jnp.float32)
    dinv_ref[...] = dinv
    hws_ref[...] = hw0 * dinv


def _tcb_body(aggp_ref, hwsp_ref, dinv_ref, b_ref, g_ref, bt_ref, Wn_ref,
              out_ref):
    aggp = aggp_ref[...]
    dinv = dinv_ref[...]
    agg = (aggp[0] + aggp[1] + hwsp_ref[...]) * dinv
    hh = (agg + b_ref[...]) * (g_ref[...] * RS) + bt_ref[...]
    h = jnp.maximum(hh, 0.0)
    out_ref[...] = jnp.dot(h, Wn_ref[...],
                           preferred_element_type=jnp.float32) * dinv


def _tcc_body(aggp_ref, hwsp_ref, dinv_ref, b_ref, g_ref, bt_ref, batch_ref,
              pW_ref, pb_ref, out_ref, s_sc, c_sc):
    i = pl.program_id(0)
    aggp = aggp_ref[...]
    agg = (aggp[0] + aggp[1] + hwsp_ref[...]) * dinv_ref[...]
    hh = (agg + b_ref[...]) * (g_ref[...] * RS) + bt_ref[...]
    br = batch_ref[0]
    gcol = lax.broadcasted_iota(jnp.int32, (G, 1), 0)
    m = (br == gcol).astype(jnp.float32)

    @pl.when(i == 0)
    def _():
        s_sc[...] = jnp.zeros((G, D), jnp.float32)
        c_sc[...] = jnp.zeros((G, 1), jnp.float32)

    s_sc[...] += jnp.dot(m, hh, preferred_element_type=jnp.float32)
    c_sc[...] += jnp.sum(m, axis=1, keepdims=True)

    @pl.when(i == GRID - 1)
    def _():
        hg = s_sc[...] / jnp.maximum(c_sc[...], 1.0)
        out_ref[...] = (jnp.dot(hg, pW_ref[...],
                                preferred_element_type=jnp.float32)
                        + pb_ref[...])


def _row_spec(w):
    return pl.BlockSpec((R, w), lambda i: (i, 0))


def _const_spec(shape):
    ndim = len(shape)
    return pl.BlockSpec(shape, lambda i: (0,) * ndim)


_AGGP_SPEC = pl.BlockSpec((2, R, D), lambda i: (0, i, 0))
_F32 = jnp.float32


def _tc_a(degp3, x, embW, embb2, W0):
    return pl.pallas_call(
        _tca_body,
        grid=(GRID,),
        in_specs=[_AGGP_SPEC,
                  _row_spec(1), _const_spec((1, D)), _const_spec((1, D)),
                  _const_spec((D, D))],
        out_specs=[_row_spec(1), _row_spec(D)],
        out_shape=[jax.ShapeDtypeStruct((N, 1), _F32),
                   jax.ShapeDtypeStruct((N, D), _F32)],
    )(degp3, x, embW, embb2, W0)


def _tc_b(aggp3, hwsp, dinv, b2, g2, bt2, Wn):
    return pl.pallas_call(
        _tcb_body,
        grid=(GRID,),
        in_specs=[_AGGP_SPEC, _row_spec(D), _row_spec(1),
                  _const_spec((1, D)), _const_spec((1, D)),
                  _const_spec((1, D)), _const_spec((D, D))],
        out_specs=_row_spec(D),
        out_shape=jax.ShapeDtypeStruct((N, D), _F32),
    )(aggp3, hwsp, dinv, b2, g2, bt2, Wn)


def _tc_c(aggp3, hwsp, dinv, b2, g2, bt2, batch3, pW, pb2):
    return pl.pallas_call(
        _tcc_body,
        grid=(GRID,),
        in_specs=[_AGGP_SPEC, _row_spec(D), _row_spec(1),
                  _const_spec((1, D)), _const_spec((1, D)),
                  _const_spec((1, D)),
                  pl.BlockSpec((1, 1, R), lambda i: (i, 0, 0)),
                  _const_spec((D, T)), _const_spec((1, T))],
        out_specs=pl.BlockSpec((G, T), lambda i: (0, 0)),
        out_shape=jax.ShapeDtypeStruct((G, T), _F32),
        scratch_shapes=[pltpu.VMEM((G, D), _F32), pltpu.VMEM((G, 1), _F32)],
    )(aggp3, hwsp, dinv, b2, g2, bt2, batch3, pW, pb2)


def kernel(x, edge_index, edge_attr, batch, emb_W, emb_b, conv_W, conv_b,
           bn_gamma, bn_beta, pred_W, pred_b):
    E = edge_index.shape[1]
    pad = EPAD - E
    pad_src = jnp.arange(pad, dtype=jnp.int32) % N
    src_f = jnp.concatenate([edge_index[0], pad_src]).reshape(TOTC, CH)
    pad_dst = N + jnp.arange(pad, dtype=jnp.int32) % (NPAD - N)
    dst_f = jnp.concatenate([edge_index[1], pad_dst]).reshape(TOTC, CH)
    dst_p = dst_f.reshape(NW, C, CH)
    ones128 = jnp.ones((CH, D), _F32)
    zacc = jnp.zeros((RPS, D), _F32)

    degp = _sc_degree(dst_p, ones128, zacc).reshape(2, NPAD, D)
    dinv, hws = _tc_a(degp, x, emb_W, emb_b.reshape(1, D), conv_W[0])

    for l in range(L - 1):
        aggp = _sc_scatter(hws, src_f, dst_f, zacc).reshape(2, NPAD, D)
        hws = _tc_b(aggp, hws, dinv, conv_b[l].reshape(1, D),
                    bn_gamma[l].reshape(1, D), bn_beta[l].reshape(1, D),
                    conv_W[l + 1])

    aggp = _sc_scatter(hws, src_f, dst_f, zacc).reshape(2, NPAD, D)
    return _tc_c(aggp, hws, dinv, conv_b[L - 1].reshape(1, D),
                 bn_gamma[L - 1].reshape(1, D), bn_beta[L - 1].reshape(1, D),
                 batch.reshape(GRID, 1, R), pred_W, pred_b.reshape(1, T))
